# Initial kernel scaffold; baseline (speedup 1.0000x reference)
import jax
import jax.numpy as jnp
from jax.experimental import pallas as pl


def kernel(graph_embedding, edge_index, W0, b0, W1, b1):
    raise NotImplementedError("write your pallas kernel here")


# speedup vs baseline: 5.3038x; 5.3038x over previous
import functools

import jax
import jax.numpy as jnp
from jax import lax
from jax.experimental import pallas as pl
from jax.experimental.pallas import tpu as pltpu
from jax.experimental.pallas import tpu_sc as plsc

N = 10000
E = 320000
D = 128

NC = 2
NS = 16
NW = NC * NS
EPW = E // NW
K = 80
NCHUNK = EPW // K
NPAD = 10240
RPS = NPAD // NS


def _mlp_body(x_ref, w0t_ref, b0_ref, w1t_ref, b1_ref, o_ref):
    h = jnp.dot(x_ref[...], w0t_ref[...], preferred_element_type=jnp.float32)
    h = h + b0_ref[...]
    h = jnp.dot(h, w1t_ref[...], preferred_element_type=jnp.float32)
    h = h + b1_ref[...]
    o_ref[...] = jnp.where(h > 0, h, jnp.exp(jnp.minimum(h, 0.0)) - 1.0)


def _mlp(x, w0t, b0, w1t, b1):
    BN = 1000
    return pl.pallas_call(
        _mlp_body,
        grid=(N // BN,),
        in_specs=[
            pl.BlockSpec((BN, D), lambda i: (i, 0)),
            pl.BlockSpec((D, D), lambda i: (0, 0)),
            pl.BlockSpec((1, D), lambda i: (0, 0)),
            pl.BlockSpec((D, D), lambda i: (0, 0)),
            pl.BlockSpec((1, D), lambda i: (0, 0)),
        ],
        out_specs=pl.BlockSpec((BN, D), lambda i: (i, 0)),
        out_shape=jax.ShapeDtypeStruct((N, D), jnp.float32),
    )(x, w0t, b0, w1t, b1)


def _sc_agg(emb, src, dst, zeros):
    mesh = plsc.VectorSubcoreMesh(core_axis_name="c", subcore_axis_name="s")

    @functools.partial(
        pl.kernel,
        out_type=jax.ShapeDtypeStruct((NC, NPAD, D), jnp.float32),
        mesh=mesh,
        scratch_types=[
            pltpu.VMEM((K,), jnp.int32),
            pltpu.VMEM((K,), jnp.int32),
            pltpu.VMEM((K, D), jnp.float32),
            pltpu.VMEM_SHARED((NPAD, D), jnp.float32),
        ],
    )
    def k(emb_hbm, src_hbm, dst_hbm, z_hbm, out_hbm, sidx, didx, rows, acc):
        c = lax.axis_index("c")
        s = lax.axis_index("s")
        wid = c * NS + s

        pltpu.sync_copy(z_hbm, acc.at[pl.ds(s * RPS, RPS)])
        plsc.subcore_barrier()

        base = wid * EPW

        @pl.loop(0, NCHUNK)
        def _(i):
            off = base + i * K
            pltpu.sync_copy(src_hbm.at[pl.ds(off, K)], sidx)
            pltpu.sync_copy(dst_hbm.at[pl.ds(off, K)], didx)
            pltpu.sync_copy(emb_hbm.at[sidx], rows)
            pltpu.sync_copy(rows, acc.at[didx], add=True)

        plsc.subcore_barrier()
        pltpu.sync_copy(
            acc.at[pl.ds(s * RPS, RPS)],
            out_hbm.at[c, pl.ds(s * RPS, RPS)],
        )

    return k(emb, src, dst, zeros)


def _add_body(p_ref, o_ref):
    o_ref[...] = p_ref[0] + p_ref[1]


def _partial_add(p):
    BN = 1000
    return pl.pallas_call(
        _add_body,
        grid=(N // BN,),
        in_specs=[pl.BlockSpec((NC, BN, D), lambda i: (0, i, 0))],
        out_specs=pl.BlockSpec((BN, D), lambda i: (i, 0)),
        out_shape=jax.ShapeDtypeStruct((N, D), jnp.float32),
    )(p)


def kernel(graph_embedding, edge_index, W0, b0, W1, b1):
    x = graph_embedding.astype(jnp.float32)
    emb = _mlp(x, W0.T, b0.reshape(1, D), W1.T, b1.reshape(1, D))
    src = edge_index[0].astype(jnp.int32)
    dst = edge_index[1].astype(jnp.int32)
    zeros = jnp.zeros((RPS, D), jnp.float32)
    partials = _sc_agg(emb, src, dst, zeros)
    return _partial_add(partials)


# speedup vs baseline: 7.2795x; 1.3725x over previous
import functools

import jax
import jax.numpy as jnp
from jax import lax
from jax.experimental import pallas as pl
from jax.experimental.pallas import tpu as pltpu
from jax.experimental.pallas import tpu_sc as plsc

N = 10000
E = 320000
D = 128

NC = 2
NS = 16
NW = NC * NS
EPW = E // NW
K = 80
NCHUNK = EPW // K
NPAD = 10240
RPS = NPAD // NS


def _mlp_body(x_ref, w0t_ref, b0_ref, w1t_ref, b1_ref, o_ref):
    h = jnp.dot(x_ref[...], w0t_ref[...], preferred_element_type=jnp.float32)
    h = h + b0_ref[...]
    h = jnp.dot(h, w1t_ref[...], preferred_element_type=jnp.float32)
    h = h + b1_ref[...]
    o_ref[...] = jnp.where(h > 0, h, jnp.exp(jnp.minimum(h, 0.0)) - 1.0)


def _mlp(x, w0t, b0, w1t, b1):
    BN = 1000
    return pl.pallas_call(
        _mlp_body,
        grid=(N // BN,),
        in_specs=[
            pl.BlockSpec((BN, D), lambda i: (i, 0)),
            pl.BlockSpec((D, D), lambda i: (0, 0)),
            pl.BlockSpec((1, D), lambda i: (0, 0)),
            pl.BlockSpec((D, D), lambda i: (0, 0)),
            pl.BlockSpec((1, D), lambda i: (0, 0)),
        ],
        out_specs=pl.BlockSpec((BN, D), lambda i: (i, 0)),
        out_shape=jax.ShapeDtypeStruct((N, D), jnp.float32),
    )(x, w0t, b0, w1t, b1)


def _sc_agg(emb, src, dst, zeros):
    mesh = plsc.VectorSubcoreMesh(core_axis_name="c", subcore_axis_name="s")

    @functools.partial(
        pl.kernel,
        out_type=jax.ShapeDtypeStruct((NC, NPAD, D), jnp.float32),
        mesh=mesh,
        scratch_types=[
            pltpu.VMEM((NCHUNK, K), jnp.int32),
            pltpu.VMEM((NCHUNK, K), jnp.int32),
            pltpu.VMEM((K, D), jnp.float32),
            pltpu.VMEM_SHARED((NPAD, D), jnp.float32),
        ],
    )
    def k(emb_hbm, src_hbm, dst_hbm, z_hbm, out_hbm, sidx, didx, rows, acc):
        c = lax.axis_index("c")
        s = lax.axis_index("s")
        wid = c * NS + s

        pltpu.sync_copy(src_hbm.at[wid], sidx)
        pltpu.sync_copy(dst_hbm.at[wid], didx)

        pltpu.sync_copy(z_hbm, acc.at[pl.ds(s * RPS, RPS)])
        plsc.subcore_barrier()

        @pl.loop(0, NCHUNK)
        def _(i):
            pltpu.sync_copy(emb_hbm.at[sidx.at[i]], rows)
            pltpu.sync_copy(rows, acc.at[didx.at[i]], add=True)

        plsc.subcore_barrier()
        pltpu.sync_copy(
            acc.at[pl.ds(s * RPS, RPS)],
            out_hbm.at[c, pl.ds(s * RPS, RPS)],
        )

    return k(emb, src, dst, zeros)


def _add_body(p_ref, o_ref):
    o_ref[...] = p_ref[0] + p_ref[1]


def _partial_add(p):
    BN = 1000
    return pl.pallas_call(
        _add_body,
        grid=(N // BN,),
        in_specs=[pl.BlockSpec((NC, BN, D), lambda i: (0, i, 0))],
        out_specs=pl.BlockSpec((BN, D), lambda i: (i, 0)),
        out_shape=jax.ShapeDtypeStruct((N, D), jnp.float32),
    )(p)


def kernel(graph_embedding, edge_index, W0, b0, W1, b1):
    x = graph_embedding.astype(jnp.float32)
    emb = _mlp(x, W0.T, b0.reshape(1, D), W1.T, b1.reshape(1, D))
    src = edge_index[0].astype(jnp.int32).reshape(NW, NCHUNK, K)
    dst = edge_index[1].astype(jnp.int32).reshape(NW, NCHUNK, K)
    zeros = jnp.zeros((RPS, D), jnp.float32)
    partials = _sc_agg(emb, src, dst, zeros)
    return _partial_add(partials)


# speedup vs baseline: 10.4138x; 1.4306x over previous
import functools

import jax
import jax.numpy as jnp
from jax import lax
from jax.experimental import pallas as pl
from jax.experimental.pallas import tpu as pltpu
from jax.experimental.pallas import tpu_sc as plsc

N = 10000
E = 320000
D = 128

NC = 2
NS = 16
NW = NC * NS
EPW = E // NW
K = 100
NCHUNK = EPW // K
NPAIR = NCHUNK // 2
NPAD = 10240
RPS = NPAD // NS


def _mlp_body(x_ref, w0t_ref, b0_ref, w1t_ref, b1_ref, o_ref):
    h = jnp.dot(x_ref[...], w0t_ref[...], preferred_element_type=jnp.float32)
    h = h + b0_ref[...]
    h = jnp.dot(h, w1t_ref[...], preferred_element_type=jnp.float32)
    h = h + b1_ref[...]
    o_ref[...] = jnp.where(h > 0, h, jnp.exp(jnp.minimum(h, 0.0)) - 1.0)


def _mlp(x, w0t, b0, w1t, b1):
    BN = 1000
    return pl.pallas_call(
        _mlp_body,
        grid=(N // BN,),
        in_specs=[
            pl.BlockSpec((BN, D), lambda i: (i, 0)),
            pl.BlockSpec((D, D), lambda i: (0, 0)),
            pl.BlockSpec((1, D), lambda i: (0, 0)),
            pl.BlockSpec((D, D), lambda i: (0, 0)),
            pl.BlockSpec((1, D), lambda i: (0, 0)),
        ],
        out_specs=pl.BlockSpec((BN, D), lambda i: (i, 0)),
        out_shape=jax.ShapeDtypeStruct((N, D), jnp.float32),
    )(x, w0t, b0, w1t, b1)


def _sc_agg(emb, idx, zeros):
    mesh = plsc.VectorSubcoreMesh(core_axis_name="c", subcore_axis_name="s")

    @functools.partial(
        pl.kernel,
        out_type=jax.ShapeDtypeStruct((NC, NPAD, D), jnp.float32),
        mesh=mesh,
        scratch_types=[
            pltpu.VMEM((2, 2, K), jnp.int32),
            pltpu.VMEM((2, K, D), jnp.float32),
            pltpu.VMEM_SHARED((NPAD, D), jnp.float32),
            pltpu.SemaphoreType.DMA,
            pltpu.SemaphoreType.DMA,
            pltpu.SemaphoreType.DMA,
            pltpu.SemaphoreType.DMA,
        ],
    )
    def k(emb_hbm, idx_hbm, z_hbm, out_hbm, ib, rows, acc, si0, si1, sg0, sg1):
        c = lax.axis_index("c")
        s = lax.axis_index("s")
        wid = c * NS + s

        pltpu.async_copy(idx_hbm.at[wid, 0], ib.at[0], si0)
        pltpu.sync_copy(z_hbm, acc.at[pl.ds(s * RPS, RPS)])
        plsc.subcore_barrier()

        @pl.loop(0, NPAIR)
        def _(j):
            a = 2 * j
            pltpu.async_copy(idx_hbm.at[wid, a + 1], ib.at[1], si1)
            pltpu.make_async_copy(idx_hbm.at[wid, a], ib.at[0], si0).wait()
            g0 = pltpu.async_copy(emb_hbm.at[ib.at[0, 0]], rows.at[0], sg0)
            pltpu.make_async_copy(idx_hbm.at[wid, a + 1], ib.at[1], si1).wait()
            g1 = pltpu.async_copy(emb_hbm.at[ib.at[1, 0]], rows.at[1], sg1)
            g0.wait()
            pltpu.sync_copy(rows.at[0], acc.at[ib.at[0, 1]], add=True)

            @pl.when(j < NPAIR - 1)
            def _():
                pltpu.async_copy(idx_hbm.at[wid, a + 2], ib.at[0], si0)

            g1.wait()
            pltpu.sync_copy(rows.at[1], acc.at[ib.at[1, 1]], add=True)

        plsc.subcore_barrier()
        pltpu.sync_copy(
            acc.at[pl.ds(s * RPS, RPS)],
            out_hbm.at[c, pl.ds(s * RPS, RPS)],
        )

    return k(emb, idx, zeros)


def _add_body(p_ref, o_ref):
    o_ref[...] = p_ref[0] + p_ref[1]


def _partial_add(p):
    BN = 1000
    return pl.pallas_call(
        _add_body,
        grid=(N // BN,),
        in_specs=[pl.BlockSpec((NC, BN, D), lambda i: (0, i, 0))],
        out_specs=pl.BlockSpec((BN, D), lambda i: (i, 0)),
        out_shape=jax.ShapeDtypeStruct((N, D), jnp.float32),
    )(p)


def kernel(graph_embedding, edge_index, W0, b0, W1, b1):
    x = graph_embedding.astype(jnp.float32)
    emb = _mlp(x, W0.T, b0.reshape(1, D), W1.T, b1.reshape(1, D))
    idx = edge_index.astype(jnp.int32).reshape(2, NW, NCHUNK, K)
    idx = jnp.transpose(idx, (1, 2, 0, 3))
    zeros = jnp.zeros((RPS, D), jnp.float32)
    partials = _sc_agg(emb, idx, zeros)
    return _partial_add(partials)


# speedup vs baseline: 13.3355x; 1.2806x over previous
import functools

import jax
import jax.numpy as jnp
from jax import lax
from jax.experimental import pallas as pl
from jax.experimental.pallas import tpu as pltpu
from jax.experimental.pallas import tpu_sc as plsc

N = 10000
E = 320000
D = 128

NC = 2
NS = 16
NW = NC * NS
EPW = E // NW
K = 80
NCHUNK = EPW // K
NSLOT = 4
NPAD = 10112
RPS = NPAD // NS


def _mlp_body(x_ref, w0t_ref, b0_ref, w1t_ref, b1_ref, o_ref):
    h = jnp.dot(x_ref[...], w0t_ref[...], preferred_element_type=jnp.float32)
    h = h + b0_ref[...]
    h = jnp.dot(h, w1t_ref[...], preferred_element_type=jnp.float32)
    h = h + b1_ref[...]
    o_ref[...] = jnp.where(h > 0, h, jnp.exp(jnp.minimum(h, 0.0)) - 1.0)


def _mlp(x, w0t, b0, w1t, b1):
    BN = 1000
    return pl.pallas_call(
        _mlp_body,
        grid=(N // BN,),
        in_specs=[
            pl.BlockSpec((BN, D), lambda i: (i, 0)),
            pl.BlockSpec((D, D), lambda i: (0, 0)),
            pl.BlockSpec((1, D), lambda i: (0, 0)),
            pl.BlockSpec((D, D), lambda i: (0, 0)),
            pl.BlockSpec((1, D), lambda i: (0, 0)),
        ],
        out_specs=pl.BlockSpec((BN, D), lambda i: (i, 0)),
        out_shape=jax.ShapeDtypeStruct((N, D), jnp.float32),
    )(x, w0t, b0, w1t, b1)


def _sc_agg(emb, idx, zeros):
    mesh = plsc.VectorSubcoreMesh(core_axis_name="c", subcore_axis_name="s")

    @functools.partial(
        pl.kernel,
        out_type=jax.ShapeDtypeStruct((NC, NPAD, D), jnp.float32),
        mesh=mesh,
        scratch_types=[
            pltpu.VMEM((NSLOT, 2, K), jnp.int32),
            pltpu.VMEM((NSLOT, K, D), jnp.float32),
            pltpu.VMEM_SHARED((NPAD, D), jnp.float32),
        ] + [pltpu.SemaphoreType.DMA] * (2 * NSLOT),
    )
    def k(emb_hbm, idx_hbm, z_hbm, out_hbm, ib, rows, acc, *sems):
        si = sems[:NSLOT]
        sg = sems[NSLOT:]
        c = lax.axis_index("c")
        s = lax.axis_index("s")
        wid = c * NS + s

        for r in range(NSLOT):
            pltpu.async_copy(idx_hbm.at[wid, r], ib.at[r], si[r])
        pltpu.sync_copy(z_hbm, acc.at[pl.ds(s * RPS, RPS)])
        plsc.subcore_barrier()

        for r in range(NSLOT - 1):
            pltpu.make_async_copy(idx_hbm.at[wid, r], ib.at[r], si[r]).wait()
            pltpu.async_copy(emb_hbm.at[ib.at[r, 0]], rows.at[r], sg[r])

        @pl.loop(0, NCHUNK // NSLOT)
        def _(j):
            c0 = NSLOT * j
            for r in range(NSLOT):
                ch = c0 + r
                r3 = (r + NSLOT - 1) % NSLOT
                pltpu.make_async_copy(
                    emb_hbm.at[ib.at[r, 0]], rows.at[r], sg[r]).wait()
                pltpu.sync_copy(rows.at[r], acc.at[ib.at[r, 1]], add=True)

                @pl.when(ch + NSLOT < NCHUNK)
                def _():
                    pltpu.async_copy(idx_hbm.at[wid, ch + NSLOT], ib.at[r], si[r])

                @pl.when(ch + NSLOT - 1 < NCHUNK)
                def _():
                    pltpu.make_async_copy(
                        idx_hbm.at[wid, ch + NSLOT - 1], ib.at[r3], si[r3]).wait()
                    pltpu.async_copy(emb_hbm.at[ib.at[r3, 0]], rows.at[r3], sg[r3])

        for r in range(NCHUNK % NSLOT):
            pltpu.make_async_copy(
                emb_hbm.at[ib.at[r, 0]], rows.at[r], sg[r]).wait()
            pltpu.sync_copy(rows.at[r], acc.at[ib.at[r, 1]], add=True)

        plsc.subcore_barrier()
        pltpu.sync_copy(
            acc.at[pl.ds(s * RPS, RPS)],
            out_hbm.at[c, pl.ds(s * RPS, RPS)],
        )

    return k(emb, idx, zeros)


def _add_body(p_ref, o_ref):
    o_ref[...] = p_ref[0] + p_ref[1]


def _partial_add(p):
    BN = 1000
    return pl.pallas_call(
        _add_body,
        grid=(N // BN,),
        in_specs=[pl.BlockSpec((NC, BN, D), lambda i: (0, i, 0))],
        out_specs=pl.BlockSpec((BN, D), lambda i: (i, 0)),
        out_shape=jax.ShapeDtypeStruct((N, D), jnp.float32),
    )(p)


def kernel(graph_embedding, edge_index, W0, b0, W1, b1):
    x = graph_embedding.astype(jnp.float32)
    emb = _mlp(x, W0.T, b0.reshape(1, D), W1.T, b1.reshape(1, D))
    idx = edge_index.astype(jnp.int32).reshape(2, NW, NCHUNK, K)
    idx = jnp.transpose(idx, (1, 2, 0, 3))
    zeros = jnp.zeros((RPS, D), jnp.float32)
    partials = _sc_agg(emb, idx, zeros)
    return _partial_add(partials)
